# baseline (speedup 1.0000x reference)
import jax
import jax.numpy as jnp
from jax import lax
from jax.experimental import pallas as pl
from jax.experimental.pallas import tpu as pltpu
from jax.experimental.pallas import tpu_sc as plsc

D_A = 64
K_CLIP = 64
L_Q = 2048
L_K = 2048

_INFO = plsc.get_sparse_core_info()
NC = _INFO.num_cores
NS = _INFO.num_subcores
LANES = _INFO.num_lanes
NW = NC * NS
ROWS_PER_W = L_Q // NW
COL_HALF = L_K // 2
GCHUNK = 128
WIN_ROWS = COL_HALF + ROWS_PER_W - 1
WIN_PAD = ((WIN_ROWS + GCHUNK - 1) // GCHUNK) * GCHUNK
FIRE = 16


def _sc_body(table_hbm, delta_hbm, out_hbm, win_v, idx_v, delta_v, gsem, osem):
    cid = lax.axis_index("c")
    sid = lax.axis_index("s")
    pltpu.sync_copy(delta_hbm, delta_v)
    dvec = delta_v[...]
    lanes = lax.iota(jnp.int32, LANES)

    row0 = (cid * NS + sid) * ROWS_PER_W
    for h in range(2):
        j0 = h * COL_HALF
        t0 = (L_Q - 1) - (row0 + ROWS_PER_W - 1) + j0
        for c in range(WIN_PAD // GCHUNK):
            for v in range(GCHUNK // LANES):
                t = t0 + c * GCHUNK + v * LANES + lanes
                idx = jnp.clip(t - (L_Q - 1) + dvec, -K_CLIP, K_CLIP) + K_CLIP
                idx_v[pl.ds(v * LANES, LANES)] = idx
            pltpu.async_copy(
                table_hbm.at[idx_v], win_v.at[pl.ds(c * GCHUNK, GCHUNK)], gsem
            ).wait()
        for b in range(0, ROWS_PER_W, FIRE):
            cps = []
            for r in range(b, b + FIRE):
                cps.append(
                    pltpu.async_copy(
                        win_v.at[pl.ds(ROWS_PER_W - 1 - r, COL_HALF)],
                        out_hbm.at[row0 + r, pl.ds(j0, COL_HALF)],
                        osem,
                    )
                )
            for cp in cps:
                cp.wait()


def kernel(length_query, length_key, position_embeddings):
    delta = jnp.full(
        (LANES,),
        jnp.asarray(length_key, jnp.int32) - jnp.asarray(length_query, jnp.int32),
        jnp.int32,
    )
    run = pl.kernel(
        _sc_body,
        out_type=jax.ShapeDtypeStruct((L_Q, L_K, D_A), jnp.float32),
        mesh=plsc.VectorSubcoreMesh(core_axis_name="c", subcore_axis_name="s"),
        scratch_types=[
            pltpu.VMEM((WIN_PAD, D_A), jnp.float32),
            pltpu.VMEM((GCHUNK,), jnp.int32),
            pltpu.VMEM((LANES,), jnp.int32),
            pltpu.SemaphoreType.DMA,
            pltpu.SemaphoreType.DMA,
        ],
        compiler_params=pltpu.CompilerParams(use_tc_tiling_on_sc=False),
    )
    return run(position_embeddings.astype(jnp.float32), delta)


# speedup vs baseline: 1.4200x; 1.4200x over previous
import jax
import jax.numpy as jnp
from jax import lax
from jax.experimental import pallas as pl
from jax.experimental.pallas import tpu as pltpu
from jax.experimental.pallas import tpu_sc as plsc

D_A = 64
K_CLIP = 64
L_Q = 2048
L_K = 2048
E_ROWS = 4096

_INFO = plsc.get_sparse_core_info()
NC = _INFO.num_cores
NS = _INFO.num_subcores
LANES = _INFO.num_lanes
NW = NC * NS
ROWS_PER_W = L_Q // NW
E_PER_S = E_ROWS // NS
GCHUNK = 128
FIRE = 16


def _sc_body(table_hbm, delta_hbm, out_hbm, e_spmem, idx_v, rows_v, delta_v, gsem, osem):
    cid = lax.axis_index("c")
    sid = lax.axis_index("s")
    pltpu.sync_copy(delta_hbm, delta_v)
    dvec = delta_v[...]
    lanes = lax.iota(jnp.int32, LANES)

    for rnd in range(E_PER_S // GCHUNK):
        base = sid * E_PER_S + rnd * GCHUNK
        for c in range(GCHUNK // LANES):
            t = base + c * LANES + lanes
            idx = jnp.clip(t - (L_Q - 1) + dvec, -K_CLIP, K_CLIP) + K_CLIP
            idx_v[pl.ds(c * LANES, LANES)] = idx
        pltpu.async_copy(table_hbm.at[idx_v], rows_v, gsem).wait()
        pltpu.sync_copy(rows_v, e_spmem.at[pl.ds(base, GCHUNK)])
    plsc.subcore_barrier()

    row0 = (cid * NS + sid) * ROWS_PER_W
    for b in range(0, ROWS_PER_W, FIRE):
        cps = []
        for r in range(b, b + FIRE):
            i = row0 + r
            start = (L_Q - 1) - i
            cps.append(
                pltpu.async_copy(
                    e_spmem.at[pl.ds(start, L_K)], out_hbm.at[i], osem
                )
            )
        for cp in cps:
            cp.wait()


def kernel(length_query, length_key, position_embeddings):
    delta = jnp.full(
        (LANES,),
        jnp.asarray(length_key, jnp.int32) - jnp.asarray(length_query, jnp.int32),
        jnp.int32,
    )
    run = pl.kernel(
        _sc_body,
        out_type=jax.ShapeDtypeStruct((L_Q, L_K, D_A), jnp.float32),
        mesh=plsc.VectorSubcoreMesh(core_axis_name="c", subcore_axis_name="s"),
        scratch_types=[
            pltpu.VMEM_SHARED((E_ROWS, D_A), jnp.float32),
            pltpu.VMEM((GCHUNK,), jnp.int32),
            pltpu.VMEM((GCHUNK, D_A), jnp.float32),
            pltpu.VMEM((LANES,), jnp.int32),
            pltpu.SemaphoreType.DMA,
            pltpu.SemaphoreType.DMA,
        ],
        compiler_params=pltpu.CompilerParams(use_tc_tiling_on_sc=False),
    )
    return run(position_embeddings.astype(jnp.float32), delta)


# speedup vs baseline: 4.4779x; 3.1534x over previous
import jax
import jax.numpy as jnp
from jax import lax
from jax.experimental import pallas as pl
from jax.experimental.pallas import tpu as pltpu
from jax.experimental.pallas import tpu_sc as plsc

D_A = 64
K_CLIP = 64
L_Q = 2048
L_K = 2048

_INFO = plsc.get_sparse_core_info()
NC = _INFO.num_cores
NS = _INFO.num_subcores
LANES = _INFO.num_lanes
DT = D_A // 8
JT = L_K // 128
PHASES = 128
M_PER_PHASE = (L_Q // NC) // PHASES
KTILES = M_PER_PHASE - 1 + JT
N_DTQ = 4
PG = NS // N_DTQ
PHI_PER_SUB = PHASES // PG
EXT_W = 224
EXT_LO = 32
EXT_MAX = EXT_W - LANES


def _sc_body(ext_hbm, delta_hbm, out_hbm, ts_spmem, buf_v, ext_v, delta_v):
    cid = lax.axis_index("c")
    sid = lax.axis_index("s")
    pltpu.sync_copy(delta_hbm, delta_v)
    dsc = delta_v[...][0]
    pltpu.sync_copy(ext_hbm, ext_v)
    pg = sid // N_DTQ
    dtq = sid % N_DTQ
    core_w0 = (1 - cid) * (L_Q // NC)
    p_off = dsc - (L_Q - 1 - K_CLIP) + EXT_LO

    def unit_body(u, _):
        phi = pg * PHI_PER_SUB + u
        v_base = core_w0 + phi

        def k_body(k, _):
            def dp_body(dp, _):
                for dt2 in range(2):
                    d = dtq * 16 + dt2 * 8 + dp
                    for ch in range(128 // LANES):
                        t0 = v_base + k * 128 + ch * LANES
                        p0 = jnp.clip(t0 + p_off, 0, EXT_MAX)
                        buf_v[dt2, k, dp, pl.ds(ch * LANES, LANES)] = (
                            ext_v[d, pl.ds(p0, LANES)]
                        )
                return 0
            lax.fori_loop(0, 8, dp_body, 0)
            return 0
        lax.fori_loop(0, KTILES, k_body, 0)

        pltpu.sync_copy(buf_v, ts_spmem.at[sid])

        def m_body(m, _):
            i = (L_Q - 1) - (v_base + PHASES * m)
            for dt2 in range(2):
                pltpu.sync_copy(
                    ts_spmem.at[sid, dt2, pl.ds(m, JT)],
                    out_hbm.at[i, dtq * 2 + dt2],
                )
            return 0
        lax.fori_loop(0, M_PER_PHASE, m_body, 0)
        return 0

    lax.fori_loop(0, PHI_PER_SUB, unit_body, 0)


def kernel(length_query, length_key, position_embeddings):
    table = position_embeddings.astype(jnp.float32)
    lo = table[0][:, None]
    hi = table[2 * K_CLIP][:, None]
    ext = jnp.concatenate(
        [
            jnp.tile(lo, (1, EXT_LO)),
            table.T,
            jnp.tile(hi, (1, EXT_W - EXT_LO - (2 * K_CLIP + 1))),
        ],
        axis=1,
    )
    delta = jnp.full(
        (LANES,),
        jnp.asarray(length_key, jnp.int32) - jnp.asarray(length_query, jnp.int32),
        jnp.int32,
    )
    run = pl.kernel(
        _sc_body,
        out_type=jax.ShapeDtypeStruct((L_Q, DT, JT, 8, 128), jnp.float32),
        mesh=plsc.VectorSubcoreMesh(core_axis_name="c", subcore_axis_name="s"),
        scratch_types=[
            pltpu.VMEM_SHARED((NS, 2, KTILES, 8, 128), jnp.float32),
            pltpu.VMEM((2, KTILES, 8, 128), jnp.float32),
            pltpu.VMEM((D_A, EXT_W), jnp.float32),
            pltpu.VMEM((LANES,), jnp.int32),
        ],
        compiler_params=pltpu.CompilerParams(use_tc_tiling_on_sc=False),
    )
    tiled = run(ext, delta)
    return tiled.transpose(0, 2, 4, 1, 3).reshape(L_Q, L_K, D_A)


# speedup vs baseline: 7.1639x; 1.5998x over previous
import jax
import jax.numpy as jnp
from jax import lax
from jax.experimental import pallas as pl
from jax.experimental.pallas import tpu as pltpu
from jax.experimental.pallas import tpu_sc as plsc

D_A = 64
K_CLIP = 64
L_Q = 2048
L_K = 2048

_INFO = plsc.get_sparse_core_info()
NC = _INFO.num_cores
NS = _INFO.num_subcores
LANES = _INFO.num_lanes
DT = D_A // 8
JT = L_K // 128
PHASES = 128
M_PER_PHASE = (L_Q // NC) // PHASES
KTILES = M_PER_PHASE - 1 + JT
N_DTQ = 4
PG = NS // N_DTQ
PHI_PER_SUB = PHASES // PG
EXT_W = 224
EXT_LO = 32
EXT_MAX = EXT_W - LANES


def _sc_body(ext_hbm, delta_hbm, out_hbm, ts_spmem, buf_v, ext_v, delta_v, sem):
    cid = lax.axis_index("c")
    sid = lax.axis_index("s")
    pltpu.sync_copy(delta_hbm, delta_v)
    dsc = delta_v[...][0]
    pltpu.sync_copy(ext_hbm, ext_v)
    pg = sid // N_DTQ
    dtq = sid % N_DTQ
    core_w0 = (1 - cid) * (L_Q // NC)
    p_off = dsc - (L_Q - 1 - K_CLIP) + EXT_LO

    def drain_emits():
        for _ in range(2 * M_PER_PHASE):
            pltpu.make_async_copy(
                ts_spmem.at[sid, 0, pl.ds(0, JT)], out_hbm.at[0, 0], sem
            ).wait()

    def unit_body(u, _):
        phi = pg * PHI_PER_SUB + u
        v_base = core_w0 + phi

        def k_body(k, _):
            def dp_body(dp, _):
                for dt2 in range(2):
                    d = dtq * 16 + dt2 * 8 + dp
                    for ch in range(128 // LANES):
                        t0 = v_base + k * 128 + ch * LANES
                        p0 = jnp.clip(t0 + p_off, 0, EXT_MAX)
                        buf_v[dt2, k, dp, pl.ds(ch * LANES, LANES)] = (
                            ext_v[d, pl.ds(p0, LANES)]
                        )
                return 0
            lax.fori_loop(0, 8, dp_body, 0)
            return 0
        lax.fori_loop(0, KTILES, k_body, 0)

        @pl.when(u > 0)
        def _():
            drain_emits()

        pltpu.sync_copy(buf_v, ts_spmem.at[sid])

        def m_body(m, _):
            i = (L_Q - 1) - (v_base + PHASES * m)
            for dt2 in range(2):
                pltpu.async_copy(
                    ts_spmem.at[sid, dt2, pl.ds(m, JT)],
                    out_hbm.at[i, dtq * 2 + dt2],
                    sem,
                )
            return 0
        lax.fori_loop(0, M_PER_PHASE, m_body, 0)
        return 0

    lax.fori_loop(0, PHI_PER_SUB, unit_body, 0)
    drain_emits()


def kernel(length_query, length_key, position_embeddings):
    table = position_embeddings.astype(jnp.float32)
    lo = table[0][:, None]
    hi = table[2 * K_CLIP][:, None]
    ext = jnp.concatenate(
        [
            jnp.tile(lo, (1, EXT_LO)),
            table.T,
            jnp.tile(hi, (1, EXT_W - EXT_LO - (2 * K_CLIP + 1))),
        ],
        axis=1,
    )
    delta = jnp.full(
        (LANES,),
        jnp.asarray(length_key, jnp.int32) - jnp.asarray(length_query, jnp.int32),
        jnp.int32,
    )
    run = pl.kernel(
        _sc_body,
        out_type=jax.ShapeDtypeStruct((L_Q, DT, JT, 8, 128), jnp.float32),
        mesh=plsc.VectorSubcoreMesh(core_axis_name="c", subcore_axis_name="s"),
        scratch_types=[
            pltpu.VMEM_SHARED((NS, 2, KTILES, 8, 128), jnp.float32),
            pltpu.VMEM((2, KTILES, 8, 128), jnp.float32),
            pltpu.VMEM((D_A, EXT_W), jnp.float32),
            pltpu.VMEM((LANES,), jnp.int32),
            pltpu.SemaphoreType.DMA,
        ],
        compiler_params=pltpu.CompilerParams(use_tc_tiling_on_sc=False),
    )
    tiled = run(ext, delta)
    return tiled.transpose(0, 2, 4, 1, 3).reshape(L_Q, L_K, D_A)
